# baseline (speedup 1.0000x reference)
import functools

import jax
import jax.numpy as jnp
from jax import lax
from jax.experimental import pallas as pl
from jax.experimental.pallas import tpu as pltpu
from jax.experimental.pallas import tpu_sc as plsc

N = 10000
D = 128
NF = 125
K = 8
NB = 4
L = 16
NC = 2
NS = 16
NW = NC * NS
QPW = 320
QPAD = NW * QPW
G = 8
NGROUPS = QPW // G

INF = 3.0e38
PEN = 1e10


def _scalar_i32(vec, j):
    lanes = lax.iota(jnp.int32, L)
    vf = jnp.where(lanes == j, vec, jnp.int32(0)).astype(jnp.float32)
    return jnp.sum(vf).astype(jnp.int32)


def _sc_body(kx_hbm, ky_hbm, kz_hbm, ks_hbm, kb_hbm,
             qx_hbm, qy_hbm, qz_hbm, qs_hbm, qb_hbm,
             table_hbm, f2_hbm, bounds_hbm,
             out_hbm,
             kx_v, ky_v, kz_v, ks_v, kb_v,
             qx_v, qy_v, qz_v, qs_v, qb_v,
             bounds_v, dist_buf, idx_buf, rows_v, f2_v, out_v, sem):
    wid = lax.axis_index("s") * NC + lax.axis_index("c")
    base = wid * QPW

    pltpu.sync_copy(kx_hbm, kx_v)
    pltpu.sync_copy(ky_hbm, ky_v)
    pltpu.sync_copy(kz_hbm, kz_v)
    pltpu.sync_copy(ks_hbm, ks_v)
    pltpu.sync_copy(kb_hbm, kb_v)
    pltpu.sync_copy(qx_hbm.at[pl.ds(base, QPW)], qx_v)
    pltpu.sync_copy(qy_hbm.at[pl.ds(base, QPW)], qy_v)
    pltpu.sync_copy(qz_hbm.at[pl.ds(base, QPW)], qz_v)
    pltpu.sync_copy(qs_hbm.at[pl.ds(base, QPW)], qs_v)
    pltpu.sync_copy(qb_hbm.at[pl.ds(base, QPW)], qb_v)
    pltpu.sync_copy(bounds_hbm.at[pl.ds(wid * L, L)], bounds_v)
    pltpu.sync_copy(f2_hbm.at[pl.ds(base, QPW)], f2_v)

    lanes = lax.iota(jnp.int32, L)
    lane_lt8 = lanes < K
    perm = jnp.where(lane_lt8, jnp.int32(K - 1) - lanes, jnp.int32(L - 1))

    def init_body(i, _):
        idx_buf[pl.ds(i * L, L)] = jnp.zeros((L,), jnp.int32)
        return 0

    lax.fori_loop(0, QPW, init_body, 0)

    bvec = bounds_v[...]

    def merge(dist, iv, R, Ridx):
        S, Sidx = plsc.sort_key_val(dist, iv)
        P = jnp.take_along_axis(S, perm, axis=0)
        Pidx = jnp.take_along_axis(Sidx, perm, axis=0)
        cm = P < R
        M = jnp.where(cm, P, R)
        Midx = jnp.where(cm, Pidx, Ridx)
        M = jnp.where(lane_lt8, M, INF)
        return plsc.sort_key_val(M, Midx)

    def chunk_body(kbase, sp, ci, carry):
        bs = list(carry[:K])
        ids = list(carry[K:])
        off = kbase + ci * L
        kxc = kx_v[pl.ds(off, L)]
        kyc = ky_v[pl.ds(off, L)]
        kzc = kz_v[pl.ds(off, L)]
        ksc = ks_v[pl.ds(off, L)]
        kbc = kb_v[pl.ds(off, L)]
        qxs, qys, qzs, qss, qbs = sp
        dot = (qxs * kxc + qys * kyc) + qzs * kzc
        d = jnp.maximum((qss + ksc) - jnp.float32(2.0) * dot,
                        jnp.float32(0.0))
        d = d + jnp.where(kbc == qbs, jnp.float32(0.0), PEN)

        def chain(args):
            d, bs, ids = args
            bs, ids = list(bs), list(ids)
            di = lanes + off
            for s in range(K):
                c = d < bs[s]
                hi = jnp.maximum(d, bs[s])
                bs[s] = jnp.minimum(d, bs[s])
                hid = jnp.where(c, ids[s], di)
                ids[s] = jnp.where(c, di, ids[s])
                d, di = hi, hid
            return tuple(bs), tuple(ids)

        improves = jnp.any(d < bs[K - 1])
        bs, ids = lax.cond(improves, chain, lambda a: (a[1], a[2]),
                           (d, tuple(bs), tuple(ids)))
        return tuple(bs) + tuple(ids)

    def splats(q):
        qi = jnp.full((L,), q, jnp.int32)
        return (plsc.load_gather(qx_v, [qi]),
                plsc.load_gather(qy_v, [qi]),
                plsc.load_gather(qz_v, [qi]),
                plsc.load_gather(qs_v, [qi]),
                plsc.load_gather(qb_v, [qi]))

    def query_body(kbase, kchunks, q, _):
        sp = splats(q)
        init = (jnp.full((L,), INF, jnp.float32),) * K + \
               (jnp.zeros((L,), jnp.int32),) * K
        carry = lax.fori_loop(
            0, kchunks, functools.partial(chunk_body, kbase, sp), init)
        R = jnp.full((L,), INF, jnp.float32)
        Ridx = jnp.zeros((L,), jnp.int32)
        for s in range(K):
            R, Ridx = merge(carry[s], carry[K + s], R, Ridx)
        dist_buf[pl.ds(q * L, L)] = R
        idx_buf[pl.ds(q * L, L)] = Ridx
        return 0

    for b in range(NB):
        qlo = _scalar_i32(bvec, b)
        qhi = _scalar_i32(bvec, NB + b)
        kbase = _scalar_i32(bvec, 2 * NB + b)
        kchunks = _scalar_i32(bvec, 3 * NB + b)
        lax.fori_loop(qlo, qhi,
                      functools.partial(query_body, kbase, kchunks), 0)

    col_mask0 = lanes >= 3

    def q2_body(g, j, acc):
        q = g * G + j
        dv = dist_buf[pl.ds(q * L, L)]
        qglob = jnp.full((L,), base + q, jnp.int32)
        wmask = jnp.logical_and(lane_lt8, qglob < N)
        wv = jnp.where(wmask, jnp.float32(1.0) / jnp.maximum(dv, 1e-16),
                       jnp.float32(0.0))
        den = jnp.full((L,), jnp.sum(wv), jnp.float32)
        inv_den = jnp.float32(1.0) / jnp.maximum(den, jnp.float32(1e-30))
        contrib = jnp.zeros((L,), jnp.float32)
        for lcol in range(D // L):
            num = jnp.zeros((L,), jnp.float32)
            for r in range(L):
                wr = jnp.take_along_axis(
                    wv, jnp.full((L,), r, jnp.int32), axis=0)
                row = rows_v[j * L + r, pl.ds(lcol * L, L)]
                num = num + row * wr
            e = num * inv_den - f2_v[q, pl.ds(lcol * L, L)]
            if lcol == 0:
                e = jnp.where(col_mask0, e, jnp.float32(0.0))
            contrib = contrib + e * e
        return acc + contrib

    def group_body(g, acc):
        pltpu.async_copy(
            table_hbm.at[idx_buf.at[pl.ds(g * (G * L), G * L)]],
            rows_v,
            sem).wait()
        return lax.fori_loop(0, G, functools.partial(q2_body, g), acc)

    acc = lax.fori_loop(0, NGROUPS, group_body, jnp.zeros((L,), jnp.float32))
    out_v[...] = acc
    pltpu.sync_copy(out_v, out_hbm.at[pl.ds(wid * L, L)])


_mesh = None


def _get_kernel():
    mesh = plsc.VectorSubcoreMesh(core_axis_name="c", subcore_axis_name="s",
                                  num_cores=NC, num_subcores=NS)
    return pl.kernel(
        _sc_body,
        out_type=jax.ShapeDtypeStruct((NW * L,), jnp.float32),
        mesh=mesh,
        compiler_params=pltpu.CompilerParams(needs_layout_passes=False),
        scratch_types=[
            pltpu.VMEM((N,), jnp.float32),
            pltpu.VMEM((N,), jnp.float32),
            pltpu.VMEM((N,), jnp.float32),
            pltpu.VMEM((N,), jnp.float32),
            pltpu.VMEM((N,), jnp.int32),
            pltpu.VMEM((QPW,), jnp.float32),
            pltpu.VMEM((QPW,), jnp.float32),
            pltpu.VMEM((QPW,), jnp.float32),
            pltpu.VMEM((QPW,), jnp.float32),
            pltpu.VMEM((QPW,), jnp.int32),
            pltpu.VMEM((L,), jnp.int32),
            pltpu.VMEM((QPW * L,), jnp.float32),
            pltpu.VMEM((QPW * L,), jnp.int32),
            pltpu.VMEM((G * L, D), jnp.float32),
            pltpu.VMEM((QPW, D), jnp.float32),
            pltpu.VMEM((L,), jnp.float32),
            pltpu.SemaphoreType.DMA,
        ],
    )


def kernel(true_graph_x, pred_graph_x, true_batch, pred_batch):
    c1 = true_graph_x[:, :3]
    c2 = pred_graph_x[:, :3]
    c1b = lax.optimization_barrier(c1.astype(jnp.bfloat16)).astype(jnp.float32)
    c2b = lax.optimization_barrier(c2.astype(jnp.bfloat16)).astype(jnp.float32)
    ks = jnp.sum(c1 * c1, axis=1)
    qs = jnp.sum(c2 * c2, axis=1)
    kx = c1b[:, 0]
    ky = c1b[:, 1]
    kz = c1b[:, 2]
    kb = true_batch.astype(jnp.int32)
    pb = pred_batch.astype(jnp.int32)

    pad = QPAD - N
    qx = jnp.pad(c2b[:, 0], (0, pad))
    qy = jnp.pad(c2b[:, 1], (0, pad))
    qz = jnp.pad(c2b[:, 2], (0, pad))
    qsp = jnp.pad(qs, (0, pad))
    qb = jnp.pad(pb, (0, pad), constant_values=127)
    f2p = jnp.pad(pred_graph_x, ((0, pad), (0, 0)))

    batches = jnp.arange(NB, dtype=jnp.int32)
    klo = jnp.searchsorted(kb, batches, side="left").astype(jnp.int32)
    khi = jnp.searchsorted(kb, batches, side="right").astype(jnp.int32)
    small = (khi - klo) < K
    klo = jnp.where(small, 0, klo)
    khi = jnp.where(small, N, khi)
    kbase = (klo // L) * L
    kchunks = ((khi + L - 1) // L * L - kbase) // L

    qs = jnp.searchsorted(pb, batches, side="left").astype(jnp.int32)
    qe = jnp.searchsorted(pb, batches, side="right").astype(jnp.int32)
    wbase = (jnp.arange(NW, dtype=jnp.int32) * QPW)[:, None]
    qlo_w = jnp.clip(qs[None, :] - wbase, 0, QPW)
    qhi_w = jnp.clip(qe[None, :] - wbase, 0, QPW)
    bounds = jnp.concatenate(
        [qlo_w, qhi_w,
         jnp.broadcast_to(kbase[None, :], (NW, NB)),
         jnp.broadcast_to(kchunks[None, :], (NW, NB))], axis=1)
    bounds = bounds.reshape(NW * L).astype(jnp.int32)

    out = _get_kernel()(kx, ky, kz, ks, kb, qx, qy, qz, qsp, qb,
                        true_graph_x, f2p, bounds)
    return jnp.sum(out) / jnp.float32(N * NF)


# speedup vs baseline: 1.7568x; 1.7568x over previous
import functools

import jax
import jax.numpy as jnp
from jax import lax
from jax.experimental import pallas as pl
from jax.experimental.pallas import tpu as pltpu
from jax.experimental.pallas import tpu_sc as plsc

N = 10000
D = 128
NF = 125
K = 8
NB = 4
L = 16
NC = 2
NS = 16
NW = NC * NS
QPW = 320
QPAD = NW * QPW
KPAD = 10016
G = 8
NGROUPS = QPW // G

INF = 3.0e38
PEN = 1e10


def _scalar_i32(vec, j):
    lanes = lax.iota(jnp.int32, L)
    vf = jnp.where(lanes == j, vec, jnp.int32(0)).astype(jnp.float32)
    return jnp.sum(vf).astype(jnp.int32)


def _sc_body(kx_hbm, ky_hbm, kz_hbm, ks_hbm, kb_hbm,
             qx_hbm, qy_hbm, qz_hbm, qs_hbm, qb_hbm,
             table_hbm, f2_hbm, bounds_hbm,
             out_hbm,
             kx_v, ky_v, kz_v, ks_v, kb_v,
             qx_v, qy_v, qz_v, qs_v, qb_v,
             bounds_v, dist_buf, idx_buf, rows_v, f2_v, out_v, sem):
    wid = lax.axis_index("s") * NC + lax.axis_index("c")
    base = wid * QPW

    pltpu.sync_copy(kx_hbm, kx_v)
    pltpu.sync_copy(ky_hbm, ky_v)
    pltpu.sync_copy(kz_hbm, kz_v)
    pltpu.sync_copy(ks_hbm, ks_v)
    pltpu.sync_copy(kb_hbm, kb_v)
    pltpu.sync_copy(qx_hbm.at[pl.ds(base, QPW)], qx_v)
    pltpu.sync_copy(qy_hbm.at[pl.ds(base, QPW)], qy_v)
    pltpu.sync_copy(qz_hbm.at[pl.ds(base, QPW)], qz_v)
    pltpu.sync_copy(qs_hbm.at[pl.ds(base, QPW)], qs_v)
    pltpu.sync_copy(qb_hbm.at[pl.ds(base, QPW)], qb_v)
    pltpu.sync_copy(bounds_hbm.at[pl.ds(wid * L, L)], bounds_v)
    pltpu.sync_copy(f2_hbm.at[pl.ds(base, QPW)], f2_v)

    lanes = lax.iota(jnp.int32, L)
    lane_lt8 = lanes < K
    perm = jnp.where(lane_lt8, jnp.int32(K - 1) - lanes, jnp.int32(L - 1))

    def init_body(i, _):
        idx_buf[pl.ds(i * L, L)] = jnp.zeros((L,), jnp.int32)
        return 0

    lax.fori_loop(0, QPW, init_body, 0)

    bvec = bounds_v[...]

    def merge(dist, iv, R, Ridx):
        S, Sidx = plsc.sort_key_val(dist, iv)
        P = jnp.take_along_axis(S, perm, axis=0)
        Pidx = jnp.take_along_axis(Sidx, perm, axis=0)
        cm = P < R
        M = jnp.where(cm, P, R)
        Midx = jnp.where(cm, Pidx, Ridx)
        M = jnp.where(lane_lt8, M, INF)
        return plsc.sort_key_val(M, Midx)

    def chunk_body(kbase, sp, ci, carry):
        bs = list(carry[:K])
        ids = list(carry[K:])
        qxs, qys, qzs, qss, qbs = sp
        for half in range(2):
            off = kbase + ci * (2 * L) + half * L
            kxc = kx_v[pl.ds(off, L)]
            kyc = ky_v[pl.ds(off, L)]
            kzc = kz_v[pl.ds(off, L)]
            ksc = ks_v[pl.ds(off, L)]
            kbc = kb_v[pl.ds(off, L)]
            dot = (qxs * kxc + qys * kyc) + qzs * kzc
            d = jnp.maximum((qss + ksc) - jnp.float32(2.0) * dot,
                            jnp.float32(0.0))
            d = d + jnp.where(kbc == qbs, jnp.float32(0.0), PEN)
            di = lanes + off
            for s in range(K):
                c = d < bs[s]
                hi = jnp.maximum(d, bs[s])
                bs[s] = jnp.minimum(d, bs[s])
                hid = jnp.where(c, ids[s], di)
                ids[s] = jnp.where(c, di, ids[s])
                d, di = hi, hid
        return tuple(bs) + tuple(ids)

    def splats(q):
        qi = jnp.full((L,), q, jnp.int32)
        return (plsc.load_gather(qx_v, [qi]),
                plsc.load_gather(qy_v, [qi]),
                plsc.load_gather(qz_v, [qi]),
                plsc.load_gather(qs_v, [qi]),
                plsc.load_gather(qb_v, [qi]))

    def query_body(kbase, kchunks, q, _):
        sp = splats(q)
        init = (jnp.full((L,), INF, jnp.float32),) * K + \
               (jnp.zeros((L,), jnp.int32),) * K
        carry = lax.fori_loop(
            0, kchunks, functools.partial(chunk_body, kbase, sp), init)
        R = jnp.full((L,), INF, jnp.float32)
        Ridx = jnp.zeros((L,), jnp.int32)
        for s in range(K):
            R, Ridx = merge(carry[s], carry[K + s], R, Ridx)
        dist_buf[pl.ds(q * L, L)] = R
        idx_buf[pl.ds(q * L, L)] = Ridx
        return 0

    for b in range(NB):
        qlo = _scalar_i32(bvec, b)
        qhi = _scalar_i32(bvec, NB + b)
        kbase = _scalar_i32(bvec, 2 * NB + b)
        kchunks = _scalar_i32(bvec, 3 * NB + b)
        lax.fori_loop(qlo, qhi,
                      functools.partial(query_body, kbase, kchunks), 0)

    col_mask0 = lanes >= 3

    def q2_body(g, j, acc):
        q = g * G + j
        dv = dist_buf[pl.ds(q * L, L)]
        qglob = jnp.full((L,), base + q, jnp.int32)
        wmask = jnp.logical_and(lane_lt8, qglob < N)
        wv = jnp.where(wmask, jnp.float32(1.0) / jnp.maximum(dv, 1e-16),
                       jnp.float32(0.0))
        den = jnp.full((L,), jnp.sum(wv), jnp.float32)
        inv_den = jnp.float32(1.0) / jnp.maximum(den, jnp.float32(1e-30))
        contrib = jnp.zeros((L,), jnp.float32)
        for lcol in range(D // L):
            num = jnp.zeros((L,), jnp.float32)
            for r in range(L):
                wr = jnp.take_along_axis(
                    wv, jnp.full((L,), r, jnp.int32), axis=0)
                row = rows_v[j * L + r, pl.ds(lcol * L, L)]
                num = num + row * wr
            e = num * inv_den - f2_v[q, pl.ds(lcol * L, L)]
            if lcol == 0:
                e = jnp.where(col_mask0, e, jnp.float32(0.0))
            contrib = contrib + e * e
        return acc + contrib

    def group_body(g, acc):
        pltpu.async_copy(
            table_hbm.at[idx_buf.at[pl.ds(g * (G * L), G * L)]],
            rows_v,
            sem).wait()
        return lax.fori_loop(0, G, functools.partial(q2_body, g), acc)

    acc = lax.fori_loop(0, NGROUPS, group_body, jnp.zeros((L,), jnp.float32))
    out_v[...] = acc
    pltpu.sync_copy(out_v, out_hbm.at[pl.ds(wid * L, L)])


_mesh = None


def _get_kernel():
    mesh = plsc.VectorSubcoreMesh(core_axis_name="c", subcore_axis_name="s",
                                  num_cores=NC, num_subcores=NS)
    return pl.kernel(
        _sc_body,
        out_type=jax.ShapeDtypeStruct((NW * L,), jnp.float32),
        mesh=mesh,
        compiler_params=pltpu.CompilerParams(needs_layout_passes=False),
        scratch_types=[
            pltpu.VMEM((KPAD,), jnp.float32),
            pltpu.VMEM((KPAD,), jnp.float32),
            pltpu.VMEM((KPAD,), jnp.float32),
            pltpu.VMEM((KPAD,), jnp.float32),
            pltpu.VMEM((KPAD,), jnp.int32),
            pltpu.VMEM((QPW,), jnp.float32),
            pltpu.VMEM((QPW,), jnp.float32),
            pltpu.VMEM((QPW,), jnp.float32),
            pltpu.VMEM((QPW,), jnp.float32),
            pltpu.VMEM((QPW,), jnp.int32),
            pltpu.VMEM((L,), jnp.int32),
            pltpu.VMEM((QPW * L,), jnp.float32),
            pltpu.VMEM((QPW * L,), jnp.int32),
            pltpu.VMEM((G * L, D), jnp.float32),
            pltpu.VMEM((QPW, D), jnp.float32),
            pltpu.VMEM((L,), jnp.float32),
            pltpu.SemaphoreType.DMA,
        ],
    )


def kernel(true_graph_x, pred_graph_x, true_batch, pred_batch):
    c1 = true_graph_x[:, :3]
    c2 = pred_graph_x[:, :3]
    c1b = lax.optimization_barrier(c1.astype(jnp.bfloat16)).astype(jnp.float32)
    c2b = lax.optimization_barrier(c2.astype(jnp.bfloat16)).astype(jnp.float32)
    ks = jnp.sum(c1 * c1, axis=1)
    qs = jnp.sum(c2 * c2, axis=1)
    kpad = KPAD - N
    kx = jnp.pad(c1b[:, 0], (0, kpad))
    ky = jnp.pad(c1b[:, 1], (0, kpad))
    kz = jnp.pad(c1b[:, 2], (0, kpad))
    ks = jnp.pad(ks, (0, kpad), constant_values=1e30)
    kb = jnp.pad(true_batch.astype(jnp.int32), (0, kpad),
                 constant_values=126)
    pb = pred_batch.astype(jnp.int32)

    pad = QPAD - N
    qx = jnp.pad(c2b[:, 0], (0, pad))
    qy = jnp.pad(c2b[:, 1], (0, pad))
    qz = jnp.pad(c2b[:, 2], (0, pad))
    qsp = jnp.pad(qs, (0, pad))
    qb = jnp.pad(pb, (0, pad), constant_values=127)
    f2p = jnp.pad(pred_graph_x, ((0, pad), (0, 0)))

    batches = jnp.arange(NB, dtype=jnp.int32)
    kbs = kb[:N]
    klo = jnp.searchsorted(kbs, batches, side="left").astype(jnp.int32)
    khi = jnp.searchsorted(kbs, batches, side="right").astype(jnp.int32)
    small = (khi - klo) < K
    klo = jnp.where(small, 0, klo)
    khi = jnp.where(small, N, khi)
    C = 2 * L
    kbase = (klo // C) * C
    kchunks = ((khi + C - 1) // C * C - kbase) // C

    qs = jnp.searchsorted(pb, batches, side="left").astype(jnp.int32)
    qe = jnp.searchsorted(pb, batches, side="right").astype(jnp.int32)
    wbase = (jnp.arange(NW, dtype=jnp.int32) * QPW)[:, None]
    qlo_w = jnp.clip(qs[None, :] - wbase, 0, QPW)
    qhi_w = jnp.clip(qe[None, :] - wbase, 0, QPW)
    bounds = jnp.concatenate(
        [qlo_w, qhi_w,
         jnp.broadcast_to(kbase[None, :], (NW, NB)),
         jnp.broadcast_to(kchunks[None, :], (NW, NB))], axis=1)
    bounds = bounds.reshape(NW * L).astype(jnp.int32)

    out = _get_kernel()(kx, ky, kz, ks, kb, qx, qy, qz, qsp, qb,
                        true_graph_x, f2p, bounds)
    return jnp.sum(out) / jnp.float32(N * NF)


# speedup vs baseline: 1.7587x; 1.0011x over previous
import functools

import jax
import jax.numpy as jnp
from jax import lax
from jax.experimental import pallas as pl
from jax.experimental.pallas import tpu as pltpu
from jax.experimental.pallas import tpu_sc as plsc

N = 10000
D = 128
NF = 125
K = 8
NB = 4
L = 16
NC = 2
NS = 16
NW = NC * NS
QPW = 320
QPAD = NW * QPW
KPAD = 10016
G = 8
NGROUPS = QPW // G

INF = 3.0e38
PEN = 1e10


def _scalar_i32(vec, j):
    lanes = lax.iota(jnp.int32, L)
    vf = jnp.where(lanes == j, vec, jnp.int32(0)).astype(jnp.float32)
    return jnp.sum(vf).astype(jnp.int32)


def _sc_body(kx_hbm, ky_hbm, kz_hbm, ks_hbm, kb_hbm,
             qx_hbm, qy_hbm, qz_hbm, qs_hbm, qb_hbm,
             table_hbm, f2_hbm, bounds_hbm,
             out_hbm,
             kx_v, ky_v, kz_v, ks_v, kb_v,
             qx_v, qy_v, qz_v, qs_v, qb_v,
             bounds_v, dist_buf, idx_buf, rows_v, f2_v, out_v, sem):
    wid = lax.axis_index("s") * NC + lax.axis_index("c")
    base = wid * QPW

    pltpu.sync_copy(kx_hbm, kx_v)
    pltpu.sync_copy(ky_hbm, ky_v)
    pltpu.sync_copy(kz_hbm, kz_v)
    pltpu.sync_copy(ks_hbm, ks_v)
    pltpu.sync_copy(kb_hbm, kb_v)
    pltpu.sync_copy(qx_hbm.at[pl.ds(base, QPW)], qx_v)
    pltpu.sync_copy(qy_hbm.at[pl.ds(base, QPW)], qy_v)
    pltpu.sync_copy(qz_hbm.at[pl.ds(base, QPW)], qz_v)
    pltpu.sync_copy(qs_hbm.at[pl.ds(base, QPW)], qs_v)
    pltpu.sync_copy(qb_hbm.at[pl.ds(base, QPW)], qb_v)
    pltpu.sync_copy(bounds_hbm.at[pl.ds(wid * L, L)], bounds_v)
    pltpu.sync_copy(f2_hbm.at[pl.ds(base, QPW)], f2_v)

    lanes = lax.iota(jnp.int32, L)
    lane_lt8 = lanes < K
    perm = jnp.where(lane_lt8, jnp.int32(K - 1) - lanes, jnp.int32(L - 1))

    def init_body(i, _):
        idx_buf[pl.ds(i * L, L)] = jnp.zeros((L,), jnp.int32)
        return 0

    lax.fori_loop(0, QPW, init_body, 0)

    bvec = bounds_v[...]

    def merge(dist, iv, R, Ridx):
        S, Sidx = plsc.sort_key_val(dist, iv)
        P = jnp.take_along_axis(S, perm, axis=0)
        Pidx = jnp.take_along_axis(Sidx, perm, axis=0)
        cm = P < R
        M = jnp.where(cm, P, R)
        Midx = jnp.where(cm, Pidx, Ridx)
        M = jnp.where(lane_lt8, M, INF)
        return plsc.sort_key_val(M, Midx)

    def chunk_body(kbase, sp, ci, carry):
        bs = list(carry[:K])
        ids = list(carry[K:])
        qxs, qys, qzs, qss, qbs = sp
        ds_ = []
        dis = []
        for half in range(2):
            off = kbase + ci * (2 * L) + half * L
            kxc = kx_v[pl.ds(off, L)]
            kyc = ky_v[pl.ds(off, L)]
            kzc = kz_v[pl.ds(off, L)]
            ksc = ks_v[pl.ds(off, L)]
            kbc = kb_v[pl.ds(off, L)]
            dot = (qxs * kxc + qys * kyc) + qzs * kzc
            d = jnp.maximum((qss + ksc) - jnp.float32(2.0) * dot,
                            jnp.float32(0.0))
            ds_.append(d + jnp.where(kbc == qbs, jnp.float32(0.0), PEN))
            dis.append(lanes + off)
        for s in range(K):
            for h in range(2):
                c = ds_[h] < bs[s]
                hi = jnp.maximum(ds_[h], bs[s])
                bs[s] = jnp.minimum(ds_[h], bs[s])
                hid = jnp.where(c, ids[s], dis[h])
                ids[s] = jnp.where(c, dis[h], ids[s])
                ds_[h], dis[h] = hi, hid
        return tuple(bs) + tuple(ids)

    def splats(q):
        qi = jnp.full((L,), q, jnp.int32)
        return (plsc.load_gather(qx_v, [qi]),
                plsc.load_gather(qy_v, [qi]),
                plsc.load_gather(qz_v, [qi]),
                plsc.load_gather(qs_v, [qi]),
                plsc.load_gather(qb_v, [qi]))

    def query_body(kbase, kchunks, q, _):
        sp = splats(q)
        init = (jnp.full((L,), INF, jnp.float32),) * K + \
               (jnp.zeros((L,), jnp.int32),) * K
        carry = lax.fori_loop(
            0, kchunks, functools.partial(chunk_body, kbase, sp), init)
        R = jnp.full((L,), INF, jnp.float32)
        Ridx = jnp.zeros((L,), jnp.int32)
        for s in range(K):
            R, Ridx = merge(carry[s], carry[K + s], R, Ridx)
        dist_buf[pl.ds(q * L, L)] = R
        idx_buf[pl.ds(q * L, L)] = Ridx
        return 0

    for b in range(NB):
        qlo = _scalar_i32(bvec, b)
        qhi = _scalar_i32(bvec, NB + b)
        kbase = _scalar_i32(bvec, 2 * NB + b)
        kchunks = _scalar_i32(bvec, 3 * NB + b)
        lax.fori_loop(qlo, qhi,
                      functools.partial(query_body, kbase, kchunks), 0)

    col_mask0 = lanes >= 3

    def q2_body(g, j, acc):
        q = g * G + j
        dv = dist_buf[pl.ds(q * L, L)]
        qglob = jnp.full((L,), base + q, jnp.int32)
        wmask = jnp.logical_and(lane_lt8, qglob < N)
        wv = jnp.where(wmask, jnp.float32(1.0) / jnp.maximum(dv, 1e-16),
                       jnp.float32(0.0))
        den = jnp.full((L,), jnp.sum(wv), jnp.float32)
        inv_den = jnp.float32(1.0) / jnp.maximum(den, jnp.float32(1e-30))
        contrib = jnp.zeros((L,), jnp.float32)
        for lcol in range(D // L):
            num = jnp.zeros((L,), jnp.float32)
            for r in range(L):
                wr = jnp.take_along_axis(
                    wv, jnp.full((L,), r, jnp.int32), axis=0)
                row = rows_v[j * L + r, pl.ds(lcol * L, L)]
                num = num + row * wr
            e = num * inv_den - f2_v[q, pl.ds(lcol * L, L)]
            if lcol == 0:
                e = jnp.where(col_mask0, e, jnp.float32(0.0))
            contrib = contrib + e * e
        return acc + contrib

    def group_body(g, acc):
        pltpu.async_copy(
            table_hbm.at[idx_buf.at[pl.ds(g * (G * L), G * L)]],
            rows_v,
            sem).wait()
        return lax.fori_loop(0, G, functools.partial(q2_body, g), acc)

    acc = lax.fori_loop(0, NGROUPS, group_body, jnp.zeros((L,), jnp.float32))
    out_v[...] = acc
    pltpu.sync_copy(out_v, out_hbm.at[pl.ds(wid * L, L)])


_mesh = None


def _get_kernel():
    mesh = plsc.VectorSubcoreMesh(core_axis_name="c", subcore_axis_name="s",
                                  num_cores=NC, num_subcores=NS)
    return pl.kernel(
        _sc_body,
        out_type=jax.ShapeDtypeStruct((NW * L,), jnp.float32),
        mesh=mesh,
        compiler_params=pltpu.CompilerParams(needs_layout_passes=False),
        scratch_types=[
            pltpu.VMEM((KPAD,), jnp.float32),
            pltpu.VMEM((KPAD,), jnp.float32),
            pltpu.VMEM((KPAD,), jnp.float32),
            pltpu.VMEM((KPAD,), jnp.float32),
            pltpu.VMEM((KPAD,), jnp.int32),
            pltpu.VMEM((QPW,), jnp.float32),
            pltpu.VMEM((QPW,), jnp.float32),
            pltpu.VMEM((QPW,), jnp.float32),
            pltpu.VMEM((QPW,), jnp.float32),
            pltpu.VMEM((QPW,), jnp.int32),
            pltpu.VMEM((L,), jnp.int32),
            pltpu.VMEM((QPW * L,), jnp.float32),
            pltpu.VMEM((QPW * L,), jnp.int32),
            pltpu.VMEM((G * L, D), jnp.float32),
            pltpu.VMEM((QPW, D), jnp.float32),
            pltpu.VMEM((L,), jnp.float32),
            pltpu.SemaphoreType.DMA,
        ],
    )


def kernel(true_graph_x, pred_graph_x, true_batch, pred_batch):
    c1 = true_graph_x[:, :3]
    c2 = pred_graph_x[:, :3]
    c1b = lax.optimization_barrier(c1.astype(jnp.bfloat16)).astype(jnp.float32)
    c2b = lax.optimization_barrier(c2.astype(jnp.bfloat16)).astype(jnp.float32)
    ks = jnp.sum(c1 * c1, axis=1)
    qs = jnp.sum(c2 * c2, axis=1)
    kpad = KPAD - N
    kx = jnp.pad(c1b[:, 0], (0, kpad))
    ky = jnp.pad(c1b[:, 1], (0, kpad))
    kz = jnp.pad(c1b[:, 2], (0, kpad))
    ks = jnp.pad(ks, (0, kpad), constant_values=1e30)
    kb = jnp.pad(true_batch.astype(jnp.int32), (0, kpad),
                 constant_values=126)
    pb = pred_batch.astype(jnp.int32)

    pad = QPAD - N
    qx = jnp.pad(c2b[:, 0], (0, pad))
    qy = jnp.pad(c2b[:, 1], (0, pad))
    qz = jnp.pad(c2b[:, 2], (0, pad))
    qsp = jnp.pad(qs, (0, pad))
    qb = jnp.pad(pb, (0, pad), constant_values=127)
    f2p = jnp.pad(pred_graph_x, ((0, pad), (0, 0)))

    batches = jnp.arange(NB, dtype=jnp.int32)
    kbs = kb[:N]
    klo = jnp.searchsorted(kbs, batches, side="left").astype(jnp.int32)
    khi = jnp.searchsorted(kbs, batches, side="right").astype(jnp.int32)
    small = (khi - klo) < K
    klo = jnp.where(small, 0, klo)
    khi = jnp.where(small, N, khi)
    C = 2 * L
    kbase = (klo // C) * C
    kchunks = ((khi + C - 1) // C * C - kbase) // C

    qs = jnp.searchsorted(pb, batches, side="left").astype(jnp.int32)
    qe = jnp.searchsorted(pb, batches, side="right").astype(jnp.int32)
    wbase = (jnp.arange(NW, dtype=jnp.int32) * QPW)[:, None]
    qlo_w = jnp.clip(qs[None, :] - wbase, 0, QPW)
    qhi_w = jnp.clip(qe[None, :] - wbase, 0, QPW)
    bounds = jnp.concatenate(
        [qlo_w, qhi_w,
         jnp.broadcast_to(kbase[None, :], (NW, NB)),
         jnp.broadcast_to(kchunks[None, :], (NW, NB))], axis=1)
    bounds = bounds.reshape(NW * L).astype(jnp.int32)

    out = _get_kernel()(kx, ky, kz, ks, kb, qx, qy, qz, qsp, qb,
                        true_graph_x, f2p, bounds)
    return jnp.sum(out) / jnp.float32(N * NF)
